# baseline (speedup 1.0000x reference)
import jax
import jax.numpy as jnp
from jax import lax
from jax.experimental import pallas as pl
from jax.experimental.pallas import tpu as pltpu
from jax.experimental.pallas import tpu_sc as plsc

_NC = 2
_NS = 16
_NW = _NC * _NS

_D = 64
_CHUNK = 160
_NBUF = 4


def _emb_body(idx_hbm, table_hbm, out_hbm, *rest):
    idx_v = rest[0:_NBUF]
    rows_v = rest[_NBUF:2 * _NBUF]
    isem = rest[2 * _NBUF:3 * _NBUF]
    gsem = rest[3 * _NBUF:4 * _NBUF]
    wsem = rest[4 * _NBUF:5 * _NBUF]

    wid = lax.axis_index("s") * _NC + lax.axis_index("c")
    b_per_w = idx_hbm.shape[0] // _NW
    n_groups = (b_per_w // _CHUNK) // _NBUF
    base_w = wid * b_per_w

    def chunk_base(j):
        return base_w + j * _CHUNK

    def fire_rows(p):
        def gi(i, carry):
            v = idx_v[p][pl.ds(i * 16, 16)]
            for k in range(16):
                pltpu.async_copy(table_hbm.at[v[k]],
                                 rows_v[p].at[i * 16 + k], gsem[p])
            return carry
        lax.fori_loop(0, _CHUNK // 16, gi, 0)

    def drain_rows(p):
        pltpu.make_async_copy(table_hbm.at[pl.ds(0, _CHUNK)], rows_v[p],
                              gsem[p]).wait()

    for p in range(_NBUF):
        pltpu.async_copy(idx_hbm.at[pl.ds(chunk_base(p), _CHUNK)],
                         idx_v[p], isem[p])
    for p in range(_NBUF):
        pltpu.make_async_copy(idx_hbm.at[pl.ds(chunk_base(p), _CHUNK)],
                              idx_v[p], isem[p]).wait()
        fire_rows(p)

    def body(g, carry):
        for p in range(_NBUF):
            jold = (g - 1) * _NBUF + p
            jnew = g * _NBUF + p
            drain_rows(p)
            pltpu.async_copy(idx_hbm.at[pl.ds(chunk_base(jnew), _CHUNK)],
                             idx_v[p], isem[p])
            pltpu.async_copy(rows_v[p],
                             out_hbm.at[pl.ds(chunk_base(jold), _CHUNK)],
                             wsem[p])
            pltpu.make_async_copy(rows_v[p],
                                  out_hbm.at[pl.ds(chunk_base(jold), _CHUNK)],
                                  wsem[p]).wait()
            pltpu.make_async_copy(idx_hbm.at[pl.ds(chunk_base(jnew), _CHUNK)],
                                  idx_v[p], isem[p]).wait()
            fire_rows(p)
        return carry

    lax.fori_loop(1, n_groups, body, 0)

    for p in range(_NBUF):
        jold = (n_groups - 1) * _NBUF + p
        drain_rows(p)
        pltpu.async_copy(rows_v[p],
                         out_hbm.at[pl.ds(chunk_base(jold), _CHUNK)],
                         wsem[p])
    for p in range(_NBUF):
        jold = (n_groups - 1) * _NBUF + p
        pltpu.make_async_copy(rows_v[p],
                              out_hbm.at[pl.ds(chunk_base(jold), _CHUNK)],
                              wsem[p]).wait()


def kernel(input, table):
    batch, hist = input.shape
    vocab, dim = table.shape
    n = batch * hist
    idx = input.reshape(n).astype(jnp.int32)
    mesh = plsc.VectorSubcoreMesh(core_axis_name="c", subcore_axis_name="s")
    f = pl.kernel(
        _emb_body,
        out_type=jax.ShapeDtypeStruct((n, dim), jnp.float32),
        mesh=mesh,
        scratch_types=(
            [pltpu.VMEM((_CHUNK,), jnp.int32)] * _NBUF
            + [pltpu.VMEM((_CHUNK, _D), jnp.float32)] * _NBUF
            + [pltpu.SemaphoreType.DMA] * (3 * _NBUF)
        ),
    )
    out = f(idx, table)
    return out.reshape(batch, hist, dim)


# speedup vs baseline: 1.0019x; 1.0019x over previous
import jax
import jax.numpy as jnp
from jax import lax
from jax.experimental import pallas as pl
from jax.experimental.pallas import tpu as pltpu
from jax.experimental.pallas import tpu_sc as plsc

_NC = 2
_NS = 16
_NW = _NC * _NS

_D = 64
_CHUNK = 400
_NBUF = 2


def _emb_body(idx_hbm, table_hbm, out_hbm,
              idx0, idx1, rows0, rows1,
              isem0, isem1, gsem0, gsem1, wsem0, wsem1):
    idx_v = (idx0, idx1)
    rows_v = (rows0, rows1)
    isem = (isem0, isem1)
    gsem = (gsem0, gsem1)
    wsem = (wsem0, wsem1)

    wid = lax.axis_index("s") * _NC + lax.axis_index("c")
    b_per_w = idx_hbm.shape[0] // _NW
    n_groups = (b_per_w // _CHUNK) // _NBUF
    base_w = wid * b_per_w

    def chunk_base(j):
        return base_w + j * _CHUNK

    def fire_rows(p):
        def gi(i, carry):
            v = idx_v[p][pl.ds(i * 16, 16)]
            for k in range(16):
                pltpu.async_copy(table_hbm.at[v[k]],
                                 rows_v[p].at[i * 16 + k], gsem[p])
            return carry
        lax.fori_loop(0, _CHUNK // 16, gi, 0)

    def drain_rows(p):
        pltpu.make_async_copy(table_hbm.at[pl.ds(0, _CHUNK)], rows_v[p],
                              gsem[p]).wait()

    for p in range(_NBUF):
        pltpu.async_copy(idx_hbm.at[pl.ds(chunk_base(p), _CHUNK)],
                         idx_v[p], isem[p])
    for p in range(_NBUF):
        pltpu.make_async_copy(idx_hbm.at[pl.ds(chunk_base(p), _CHUNK)],
                              idx_v[p], isem[p]).wait()
        fire_rows(p)

    def body(g, carry):
        for p in range(_NBUF):
            jold = (g - 1) * _NBUF + p
            jnew = g * _NBUF + p
            drain_rows(p)
            pltpu.async_copy(idx_hbm.at[pl.ds(chunk_base(jnew), _CHUNK)],
                             idx_v[p], isem[p])
            pltpu.async_copy(rows_v[p],
                             out_hbm.at[pl.ds(chunk_base(jold), _CHUNK)],
                             wsem[p])
            pltpu.make_async_copy(rows_v[p],
                                  out_hbm.at[pl.ds(chunk_base(jold), _CHUNK)],
                                  wsem[p]).wait()
            pltpu.make_async_copy(idx_hbm.at[pl.ds(chunk_base(jnew), _CHUNK)],
                                  idx_v[p], isem[p]).wait()
            fire_rows(p)
        return carry

    lax.fori_loop(1, n_groups, body, 0)

    for p in range(_NBUF):
        jold = (n_groups - 1) * _NBUF + p
        drain_rows(p)
        pltpu.async_copy(rows_v[p],
                         out_hbm.at[pl.ds(chunk_base(jold), _CHUNK)],
                         wsem[p])
    for p in range(_NBUF):
        jold = (n_groups - 1) * _NBUF + p
        pltpu.make_async_copy(rows_v[p],
                              out_hbm.at[pl.ds(chunk_base(jold), _CHUNK)],
                              wsem[p]).wait()


def kernel(input, table):
    batch, hist = input.shape
    vocab, dim = table.shape
    n = batch * hist
    idx = input.reshape(n).astype(jnp.int32)
    mesh = plsc.VectorSubcoreMesh(core_axis_name="c", subcore_axis_name="s")
    f = pl.kernel(
        _emb_body,
        out_type=jax.ShapeDtypeStruct((n, dim), jnp.float32),
        mesh=mesh,
        scratch_types=[
            pltpu.VMEM((_CHUNK,), jnp.int32),
            pltpu.VMEM((_CHUNK,), jnp.int32),
            pltpu.VMEM((_CHUNK, _D), jnp.float32),
            pltpu.VMEM((_CHUNK, _D), jnp.float32),
            pltpu.SemaphoreType.DMA,
            pltpu.SemaphoreType.DMA,
            pltpu.SemaphoreType.DMA,
            pltpu.SemaphoreType.DMA,
            pltpu.SemaphoreType.DMA,
            pltpu.SemaphoreType.DMA,
        ],
    )
    out = f(idx, table)
    return out.reshape(batch, hist, dim)
